# baseline (speedup 1.0000x reference)
import jax
import jax.numpy as jnp
from jax.experimental import pallas as pl


def _take_mask(C, H, W):
    mk = jax.random.key(42)
    kc, ks = jax.random.split(mk)
    cm = jax.random.randint(kc, (C,), 0, 2).astype(jnp.uint8).astype(bool)
    spatial = jax.random.randint(ks, (H,), 0, 2)
    neg_idx = jnp.bitwise_not(spatial) % W
    pos_idx = spatial % W
    neg_hit = jnp.zeros((W,), dtype=bool).at[neg_idx].set(True)
    pos_hit = jnp.zeros((W,), dtype=bool).at[pos_idx].set(True)
    return pos_hit[None, :] | (~neg_hit[None, :] & cm[:, None])


def _select_body(m_ref, a_ref, b_ref, o1_ref, o2_ref):
    m = (m_ref[...] != 0.0)[None]
    a = a_ref[...]
    b = b_ref[...]
    o1_ref[...] = jnp.where(m, b, a)
    o2_ref[...] = jnp.where(m, a, b)


def kernel(lst, gui):
    N, C, H, W = lst.shape
    mask = _take_mask(C, H, W).astype(jnp.float32).reshape(C, 1, W)

    HB = 56
    while H % HB:
        HB //= 2
    grid = (N, H // HB)
    data_spec = pl.BlockSpec((1, C, HB, W), lambda n, h: (n, 0, h, 0))
    mask_spec = pl.BlockSpec((C, 1, W), lambda n, h: (0, 0, 0))

    out_lst, out_gui = pl.pallas_call(
        _select_body,
        grid=grid,
        in_specs=[mask_spec, data_spec, data_spec],
        out_specs=[data_spec, data_spec],
        out_shape=[
            jax.ShapeDtypeStruct(lst.shape, lst.dtype),
            jax.ShapeDtypeStruct(gui.shape, gui.dtype),
        ],
    )(mask, lst, gui)
    return (out_lst, out_gui)


# speedup vs baseline: 1.0057x; 1.0057x over previous
import jax
import jax.numpy as jnp
from jax.experimental import pallas as pl
from jax.experimental.pallas import tpu as pltpu


def _take_mask(C, H, W):
    mk = jax.random.key(42)
    kc, ks = jax.random.split(mk)
    cm = jax.random.randint(kc, (C,), 0, 2).astype(jnp.uint8).astype(bool)
    spatial = jax.random.randint(ks, (H,), 0, 2)
    neg_idx = jnp.bitwise_not(spatial) % W
    pos_idx = spatial % W
    neg_hit = jnp.zeros((W,), dtype=bool).at[neg_idx].set(True)
    pos_hit = jnp.zeros((W,), dtype=bool).at[pos_idx].set(True)
    return pos_hit[None, :] | (~neg_hit[None, :] & cm[:, None])


def _select_body(m_ref, a_ref, b_ref, o1_ref, o2_ref):
    m = (m_ref[...] != 0.0)[None]
    a = a_ref[...]
    b = b_ref[...]
    o1_ref[...] = jnp.where(m, b, a)
    o2_ref[...] = jnp.where(m, a, b)


def kernel(lst, gui):
    N, C, H, W = lst.shape
    mask = _take_mask(C, H, W).astype(jnp.float32).reshape(C, 1, W)

    CB = 32
    while C % CB:
        CB //= 2
    grid = (N, C // CB)
    data_spec = pl.BlockSpec((1, CB, H, W), lambda n, c: (n, c, 0, 0))
    mask_spec = pl.BlockSpec((CB, 1, W), lambda n, c: (c, 0, 0))

    out_lst, out_gui = pl.pallas_call(
        _select_body,
        grid=grid,
        in_specs=[mask_spec, data_spec, data_spec],
        out_specs=[data_spec, data_spec],
        out_shape=[
            jax.ShapeDtypeStruct(lst.shape, lst.dtype),
            jax.ShapeDtypeStruct(gui.shape, gui.dtype),
        ],
        compiler_params=pltpu.CompilerParams(
            dimension_semantics=("parallel", "parallel")),
    )(mask, lst, gui)
    return (out_lst, out_gui)
